# baseline (speedup 1.0000x reference)
import functools

import jax
import jax.numpy as jnp
from jax import lax
from jax.experimental import pallas as pl
from jax.experimental.pallas import tpu as pltpu
from jax.experimental.pallas import tpu_sc as plsc

NC, NS, L = 2, 16, 16
NW = NC * NS
F = 40
UNROLL = 4
ROW = 112
VB = 448
BPT = 7
EPS2 = 1e-24


def _rsqrt(s):
    i = lax.bitcast_convert_type(s, jnp.int32)
    y = lax.bitcast_convert_type(jnp.int32(0x5F3759DF) - (i >> 1), jnp.float32)
    hs = 0.5 * s
    y = y * (1.5 - hs * y * y)
    y = y * (1.5 - hs * y * y)
    return y


def _dot(a, b):
    return a[0] * b[0] + a[1] * b[1] + a[2] * b[2]


@functools.partial(jax.jit, static_argnames=("nf", "nv"))
def _sc_loss(coord_out, coord_gt, valid, faces_flat, nf, nv):
    nchunk = nf // F
    mesh = plsc.VectorSubcoreMesh(core_axis_name="c", subcore_axis_name="s")

    @functools.partial(
        pl.kernel,
        mesh=mesh,
        out_type=(
            jax.ShapeDtypeStruct((L, 3, nf), jnp.float32),
            jax.ShapeDtypeStruct((NC * nv, ROW), jnp.float32),
        ),
        scratch_types=[
            pltpu.VMEM((L, VB * 3), jnp.float32),
            pltpu.VMEM((L, VB * 3), jnp.float32),
            pltpu.VMEM((L, VB), jnp.float32),
            pltpu.VMEM((VB, ROW), jnp.float32),
            pltpu.VMEM((3 * F,), jnp.int32),
            pltpu.VMEM((3 * F, ROW), jnp.float32),
            pltpu.VMEM((L, 3, F), jnp.float32),
            pltpu.SemaphoreType.DMA,
            pltpu.SemaphoreType.DMA,
            pltpu.SemaphoreType.DMA,
            pltpu.SemaphoreType.DMA,
        ],
        compiler_params=pltpu.CompilerParams(
            use_tc_tiling_on_sc=False, needs_layout_passes=False
        ),
    )
    def k(co_hbm, cg_hbm, val_hbm, face_hbm, out_hbm, tbl_hbm,
          stag, stag2, stagv, tblk, idx_v, rows_v, out_v, s1, s2, s3, sem):
        cid = lax.axis_index("c")
        sid = lax.axis_index("s")
        wid = sid * NC + cid
        lane = lax.iota(jnp.int32, 16)

        def shuffle(src, n_comp, col0):
            def sh_body(v4, carry):
                for u in range(4):
                    v = 4 * v4 + u
                    for kk in range(n_comp):
                        x = plsc.load_gather(
                            src, [lane, jnp.full((16,), n_comp * v + kk, jnp.int32)])
                        tblk[v, col0 + 16 * kk:col0 + 16 * (kk + 1)] = x
                return carry
            lax.fori_loop(0, VB // 4, sh_body, 0)

        def pack_block(t, carry):
            cv = sid + t * NS
            v0 = jnp.minimum(cv * VB, nv - VB)
            d1 = pltpu.make_async_copy(co_hbm.at[:, pl.ds(v0 * 3, VB * 3)], stag, s1)
            d2 = pltpu.make_async_copy(cg_hbm.at[:, pl.ds(v0 * 3, VB * 3)], stag2, s2)
            d3 = pltpu.make_async_copy(val_hbm.at[:, pl.ds(v0, VB)], stagv, s3)
            d1.start()
            d2.start()
            d3.start()
            d1.wait()
            shuffle(stag, 3, 0)
            d2.wait()
            shuffle(stag2, 3, 48)
            d3.wait()
            shuffle(stagv, 1, 96)
            pltpu.sync_copy(tblk, tbl_hbm.at[pl.ds(cid * nv + v0, VB), :])
            return carry

        lax.fori_loop(0, BPT, pack_block, 0)
        plsc.subcore_barrier()

        my_chunks = (nchunk - wid + NW - 1) // NW
        coff = jnp.full((16,), cid * nv, jnp.int32)

        def chunk_body(t, carry):
            c = wid + t * NW
            pltpu.sync_copy(face_hbm.at[pl.ds(c * (3 * F), 3 * F)], idx_v)
            for g in range(3 * F // 16):
                plsc.addupdate(idx_v.at[pl.ds(g * 16, 16)], coff)
            pltpu.async_copy(tbl_hbm.at[idx_v], rows_v, sem).wait()

            def one_face(j):
                r0 = 3 * j
                r1 = r0 + 1
                r2 = r0 + 2

                def ld(r, kk):
                    return rows_v[r, 16 * kk:16 * (kk + 1)]

                o0 = [ld(r0, kk) for kk in range(3)]
                o1 = [ld(r1, kk) for kk in range(3)]
                o2 = [ld(r2, kk) for kk in range(3)]
                g0 = [ld(r0, 3 + kk) for kk in range(3)]
                g1 = [ld(r1, 3 + kk) for kk in range(3)]
                g2 = [ld(r2, 3 + kk) for kk in range(3)]
                m = ld(r0, 6) * ld(r1, 6) * ld(r2, 6)

                e1 = [a - b for a, b in zip(o1, o0)]
                e2 = [a - b for a, b in zip(o2, o0)]
                e3 = [a - b for a, b in zip(e2, e1)]
                h1 = [a - b for a, b in zip(g1, g0)]
                h2 = [a - b for a, b in zip(g2, g0)]
                n = [h1[1] * h2[2] - h1[2] * h2[1],
                     h1[2] * h2[0] - h1[0] * h2[2],
                     h1[0] * h2[1] - h1[1] * h2[0]]

                snc = jnp.maximum(_dot(n, n), EPS2)
                d1 = _dot(e1, n)
                d2 = _dot(e2, n)
                d3 = d2 - d1
                c1 = jnp.abs(d1) * _rsqrt(jnp.maximum(_dot(e1, e1), EPS2) * snc) * m
                c2 = jnp.abs(d2) * _rsqrt(jnp.maximum(_dot(e2, e2), EPS2) * snc) * m
                c3 = jnp.abs(d3) * _rsqrt(jnp.maximum(_dot(e3, e3), EPS2) * snc) * m

                jv = jnp.full((16,), j, jnp.int32)
                plsc.store_scatter(out_v, [lane, jnp.full((16,), 0, jnp.int32), jv], c1)
                plsc.store_scatter(out_v, [lane, jnp.full((16,), 1, jnp.int32), jv], c2)
                plsc.store_scatter(out_v, [lane, jnp.full((16,), 2, jnp.int32), jv], c3)

            def face_body(j4, carry2):
                for jj in range(UNROLL):
                    one_face(UNROLL * j4 + jj)
                return carry2

            lax.fori_loop(0, F // UNROLL, face_body, 0)
            pltpu.sync_copy(out_v, out_hbm.at[:, :, pl.ds(c * F, F)])
            return carry

        lax.fori_loop(0, my_chunks, chunk_body, 0)

    return k(coord_out, coord_gt, valid, faces_flat)


def kernel(coord_out, coord_gt, valid, face):
    B, V, D = coord_out.shape
    nf = face.shape[0]
    out, _ = _sc_loss(coord_out.reshape(B, V * D), coord_gt.reshape(B, V * D),
                      valid.reshape(B, V), face.reshape(-1), nf, V)
    return out.reshape(B, 3 * nf, 1)


# speedup vs baseline: 2.1610x; 2.1610x over previous
import functools

import jax
import jax.numpy as jnp
from jax import lax
from jax.experimental import pallas as pl
from jax.experimental.pallas import tpu as pltpu
from jax.experimental.pallas import tpu_sc as plsc

NC, NS, L = 2, 16, 16
NW = NC * NS
F = 40
UNROLL = 4
ROW = 112
EPS2 = 1e-24


def _rsqrt(s):
    i = lax.bitcast_convert_type(s, jnp.int32)
    y = lax.bitcast_convert_type(jnp.int32(0x5F3759DF) - (i >> 1), jnp.float32)
    hs = 0.5 * s
    y = y * (1.5 - hs * y * y)
    y = y * (1.5 - hs * y * y)
    return y


def _dot(a, b):
    return a[0] * b[0] + a[1] * b[1] + a[2] * b[2]


@functools.partial(jax.jit, static_argnames=("nf",))
def _sc_loss(tbl, faces_flat, nf):
    nchunk = nf // F
    mesh = plsc.VectorSubcoreMesh(core_axis_name="c", subcore_axis_name="s")

    @functools.partial(
        pl.kernel,
        mesh=mesh,
        out_type=jax.ShapeDtypeStruct((L, 3, nf), jnp.float32),
        scratch_types=[
            pltpu.VMEM((3 * F,), jnp.int32),
            pltpu.VMEM((3 * F, ROW), jnp.float32),
            pltpu.VMEM((L, 3, F), jnp.float32),
            pltpu.SemaphoreType.DMA,
        ],
        compiler_params=pltpu.CompilerParams(
            use_tc_tiling_on_sc=False, needs_layout_passes=False
        ),
    )
    def k(tbl_hbm, face_hbm, out_hbm, idx_v, rows_v, out_v, sem):
        wid = lax.axis_index("s") * NC + lax.axis_index("c")
        my_chunks = (nchunk - wid + NW - 1) // NW
        lane = lax.iota(jnp.int32, 16)

        def chunk_body(t, carry):
            c = wid + t * NW
            pltpu.sync_copy(face_hbm.at[pl.ds(c * (3 * F), 3 * F)], idx_v)
            pltpu.async_copy(tbl_hbm.at[idx_v], rows_v, sem).wait()

            @plsc.parallel_loop(0, F, 1, unroll=UNROLL)
            def one_face(j):
                r0 = 3 * j
                r1 = r0 + 1
                r2 = r0 + 2

                def ld(r, kk):
                    return rows_v[r, 16 * kk:16 * (kk + 1)]

                o0 = [ld(r0, kk) for kk in range(3)]
                o1 = [ld(r1, kk) for kk in range(3)]
                o2 = [ld(r2, kk) for kk in range(3)]
                g0 = [ld(r0, 3 + kk) for kk in range(3)]
                g1 = [ld(r1, 3 + kk) for kk in range(3)]
                g2 = [ld(r2, 3 + kk) for kk in range(3)]
                m = ld(r0, 6) * ld(r1, 6) * ld(r2, 6)

                e1 = [a - b for a, b in zip(o1, o0)]
                e2 = [a - b for a, b in zip(o2, o0)]
                e3 = [a - b for a, b in zip(e2, e1)]
                h1 = [a - b for a, b in zip(g1, g0)]
                h2 = [a - b for a, b in zip(g2, g0)]
                n = [h1[1] * h2[2] - h1[2] * h2[1],
                     h1[2] * h2[0] - h1[0] * h2[2],
                     h1[0] * h2[1] - h1[1] * h2[0]]

                snc = jnp.maximum(_dot(n, n), EPS2)
                d1 = _dot(e1, n)
                d2 = _dot(e2, n)
                d3 = d2 - d1
                c1 = jnp.abs(d1) * _rsqrt(jnp.maximum(_dot(e1, e1), EPS2) * snc) * m
                c2 = jnp.abs(d2) * _rsqrt(jnp.maximum(_dot(e2, e2), EPS2) * snc) * m
                c3 = jnp.abs(d3) * _rsqrt(jnp.maximum(_dot(e3, e3), EPS2) * snc) * m

                jv = jnp.full((16,), j, jnp.int32)
                plsc.store_scatter(out_v, [lane, jnp.full((16,), 0, jnp.int32), jv], c1)
                plsc.store_scatter(out_v, [lane, jnp.full((16,), 1, jnp.int32), jv], c2)
                plsc.store_scatter(out_v, [lane, jnp.full((16,), 2, jnp.int32), jv], c3)

            pltpu.sync_copy(out_v, out_hbm.at[:, :, pl.ds(c * F, F)])
            return carry

        lax.fori_loop(0, my_chunks, chunk_body, 0)

    return k(tbl, faces_flat)


def kernel(coord_out, coord_gt, valid, face):
    B, V, D = coord_out.shape
    nf = face.shape[0]
    tbl = jnp.concatenate(
        [
            coord_out.transpose(1, 2, 0).reshape(V, D * B),
            coord_gt.transpose(1, 2, 0).reshape(V, D * B),
            valid[:, :, 0].T,
        ],
        axis=1,
    )
    out = _sc_loss(tbl, face.reshape(-1), nf)
    return out.reshape(B, 3 * nf, 1)
